# Initial kernel scaffold; baseline (speedup 1.0000x reference)
import jax
import jax.numpy as jnp
from jax.experimental import pallas as pl


def kernel(pdg, feat, adj, emb_table, W_in, b_in, W_h, b_h, W_out, b_out):
    raise NotImplementedError("write your pallas kernel here")


# speedup vs baseline: 1.4193x; 1.4193x over previous
import jax
import jax.numpy as jnp
from jax.experimental import pallas as pl
from jax.experimental.pallas import tpu as pltpu

_B, _N = 32, 1024
_F, _E, _U, _H = 8, 8, 32, 6
_TPAD = 128


def _gcn_body(pdg_ref, feat_ref, adj_ref, embt_ref, wf_ref, we_ref, bin_ref,
              wh_ref, bh_ref, wout_ref, bout_ref, out_ref):
    b = pl.program_id(0)
    adjm = adj_ref[0]
    featm = feat_ref[0]
    ids = pdg_ref[0]

    cols = jax.lax.broadcasted_iota(jnp.int32, (_N, _TPAD), 1)
    onehot = (cols == ids).astype(jnp.float32)
    emb = jnp.dot(onehot, embt_ref[...],
                  preferred_element_type=jnp.float32)

    h = jnp.dot(featm, wf_ref[...], preferred_element_type=jnp.float32)
    h = h + jnp.dot(emb, we_ref[...], preferred_element_type=jnp.float32)
    h = jnp.maximum(h + bin_ref[...], 0.0)

    deg = jnp.sum(adjm, axis=1, keepdims=True)
    inv = 1.0 / (deg + 1e-8)

    for i in range(_H):
        m = jnp.dot(adjm, h, preferred_element_type=jnp.float32) * inv
        h = jnp.dot(m, wh_ref[i], preferred_element_type=jnp.float32)
        h = jnp.maximum(h + bh_ref[i], 0.0)

    pooled = jnp.sum(h, axis=0, keepdims=True)
    res = jnp.dot(pooled, wout_ref[...],
                  preferred_element_type=jnp.float32) + bout_ref[...]
    out_ref[pl.ds(b, 1), :] = jnp.broadcast_to(res, (1, 128))


def kernel(pdg, feat, adj, emb_table, W_in, b_in, W_h, b_h, W_out, b_out):
    pdg3 = pdg.astype(jnp.int32).reshape(_B, _N, 1)
    embp = jnp.zeros((_TPAD, _E), jnp.float32).at[:emb_table.shape[0]].set(
        emb_table.astype(jnp.float32))
    wf = W_in[:_F]
    we = W_in[_F:]
    bin2 = b_in.reshape(1, _U)
    bh3 = b_h.reshape(_H, 1, _U)
    bout2 = b_out.reshape(1, 1)

    out = pl.pallas_call(
        _gcn_body,
        grid=(_B,),
        in_specs=[
            pl.BlockSpec((1, _N, 1), lambda b: (b, 0, 0)),
            pl.BlockSpec((1, _N, _F), lambda b: (b, 0, 0)),
            pl.BlockSpec((1, _N, _N), lambda b: (b, 0, 0)),
            pl.BlockSpec((_TPAD, _E), lambda b: (0, 0)),
            pl.BlockSpec((_F, _U), lambda b: (0, 0)),
            pl.BlockSpec((_E, _U), lambda b: (0, 0)),
            pl.BlockSpec((1, _U), lambda b: (0, 0)),
            pl.BlockSpec((_H, _U, _U), lambda b: (0, 0, 0)),
            pl.BlockSpec((_H, 1, _U), lambda b: (0, 0, 0)),
            pl.BlockSpec((_U, 1), lambda b: (0, 0)),
            pl.BlockSpec((1, 1), lambda b: (0, 0)),
        ],
        out_specs=pl.BlockSpec((_B, 128), lambda b: (0, 0)),
        out_shape=jax.ShapeDtypeStruct((_B, 128), jnp.float32),
        compiler_params=pltpu.CompilerParams(
            dimension_semantics=("arbitrary",)),
    )(pdg3, feat, adj, embp, wf, we, bin2, W_h, bh3, W_out, bout2)
    return out[:, :1]
